# baseline (speedup 1.0000x reference)
import functools
import jax
import jax.numpy as jnp
from jax import lax
from jax.experimental import pallas as pl
from jax.experimental.pallas import tpu as pltpu
from jax.experimental.pallas import tpu_sc as plsc

_SEQ = 8192
_D = 1024
_ROWS_PER_TILE = 1024
_SEQ_BLK = 1024
_STEP = 64
_KPAD = 128


def _sc_delta_body(ts_hbm, delta_hbm, tsrow_v, out_v):
    wid = lax.axis_index("s") * 2 + lax.axis_index("c")
    batch = wid // 8
    rib0 = (wid % 8) * _ROWS_PER_TILE

    pltpu.sync_copy(ts_hbm.at[pl.ds(batch * _SEQ, _SEQ)], tsrow_v)

    def _min_step(i, m):
        return jnp.minimum(m, tsrow_v[pl.ds(i * 16, 16)])

    m = lax.fori_loop(0, _SEQ // 16, _min_step,
                      jnp.full((16,), 2**31 - 1, jnp.int32))
    ii = lax.iota(jnp.int32, 16)
    dnums = lax.GatherDimensionNumbers(
        offset_dims=(), collapsed_slice_dims=(0,), start_index_map=(0,))
    for sh in (8, 4, 2, 1):
        perm = (ii + sh) & 15
        shuf = lax.gather(m, perm[:, None], dnums, slice_sizes=(1,),
                          mode=lax.GatherScatterMode.PROMISE_IN_BOUNDS)
        m = jnp.minimum(m, shuf)

    def _delta_step(i, carry):
        d = tsrow_v[pl.ds(rib0 + i * 16, 16)] - m
        out_v[pl.ds(i * 16, 16)] = jnp.clip(d, 0, 4999)
        return carry

    lax.fori_loop(0, _ROWS_PER_TILE // 16, _delta_step, 0)
    pltpu.sync_copy(out_v, delta_hbm.at[pl.ds(wid * _ROWS_PER_TILE,
                                              _ROWS_PER_TILE)])


def _sc_delta(ts_flat):
    mesh = plsc.VectorSubcoreMesh(core_axis_name="c", subcore_axis_name="s")
    k = functools.partial(
        pl.kernel,
        mesh=mesh,
        out_type=jax.ShapeDtypeStruct((ts_flat.shape[0],), jnp.int32),
        scratch_types=[
            pltpu.VMEM((_SEQ,), jnp.int32),
            pltpu.VMEM((_ROWS_PER_TILE,), jnp.int32),
        ],
    )(_sc_delta_body)
    return k(ts_flat)


def _pe_add_body(d_ref, x_ref, hicat_ref, locat_ref, o_ref):
    m = _SEQ_BLK
    d = x_ref.shape[-1]
    delta = d_ref[0, 0, :]
    hi = delta // _STEP
    lo = delta - hi * _STEP
    kio = jax.lax.broadcasted_iota(jnp.int32, (m, _KPAD), 1)
    a = (hi[:, None] == kio).astype(jnp.bfloat16)
    b = (lo[:, None] == kio).astype(jnp.bfloat16)
    hsw = jnp.dot(a, hicat_ref[...], preferred_element_type=jnp.float32)
    uv = jnp.dot(b, locat_ref[...], preferred_element_type=jnp.float32)
    pe = hsw[:, :d] * uv[:, :d] + hsw[:, d:] * uv[:, d:]
    o_ref[0, :, :] = x_ref[0, :, :] + pe


def _pair_swap(t):
    n, d = t.shape
    return t.reshape(n, d // 2, 2)[:, :, ::-1].reshape(n, d)


def kernel(x, timesteps, pos_encoding):
    b, seq, one, d = x.shape

    x3 = x.reshape(b, seq, d)
    ts_flat = timesteps.reshape(b * seq).astype(jnp.int32)

    delta = _sc_delta(ts_flat).reshape(b, 1, seq)

    n_hi = (pos_encoding.shape[0] + _STEP - 1) // _STEP
    hi_t = pos_encoding[:: _STEP]
    lo_t = pos_encoding[:_STEP]
    hi_sw = _pair_swap(hi_t)
    lo_sw = _pair_swap(lo_t)
    even = (jnp.arange(d) % 2 == 0)[None, :]
    u_t = jnp.where(even, lo_sw, lo_t)
    v_t = jnp.where(even, lo_t, -lo_sw)

    def _pad(t, rows):
        return jnp.pad(t, ((0, _KPAD - rows), (0, 0)))

    hicat = jnp.concatenate([_pad(hi_t, n_hi), _pad(hi_sw, n_hi)], axis=1)
    locat = jnp.concatenate([_pad(u_t, _STEP), _pad(v_t, _STEP)], axis=1)
    hicat = hicat.astype(jnp.bfloat16)
    locat = locat.astype(jnp.bfloat16)

    n_s = seq // _SEQ_BLK
    out = pl.pallas_call(
        _pe_add_body,
        grid=(b, n_s),
        in_specs=[
            pl.BlockSpec((1, 1, _SEQ_BLK), lambda i, j: (i, 0, j)),
            pl.BlockSpec((1, _SEQ_BLK, d), lambda i, j: (i, j, 0)),
            pl.BlockSpec((_KPAD, 2 * d), lambda i, j: (0, 0)),
            pl.BlockSpec((_KPAD, 2 * d), lambda i, j: (0, 0)),
        ],
        out_specs=pl.BlockSpec((1, _SEQ_BLK, d), lambda i, j: (i, j, 0)),
        out_shape=jax.ShapeDtypeStruct((b, seq, d), x.dtype),
    )(delta, x3, hicat, locat)
    return out.reshape(b, seq, one, d)


# speedup vs baseline: 1.0287x; 1.0287x over previous
import functools
import jax
import jax.numpy as jnp
from jax import lax
from jax.experimental import pallas as pl
from jax.experimental.pallas import tpu as pltpu
from jax.experimental.pallas import tpu_sc as plsc

_SEQ = 8192
_D = 1024
_ROWS_PER_TILE = 1024
_SEQ_BLK = 2048
_STEP = 64
_KPAD = 128


def _sc_delta_body(ts_hbm, delta_hbm, tsrow_v, out_v):
    wid = lax.axis_index("s") * 2 + lax.axis_index("c")
    batch = wid // 8
    rib0 = (wid % 8) * _ROWS_PER_TILE

    pltpu.sync_copy(ts_hbm.at[pl.ds(batch * _SEQ, _SEQ)], tsrow_v)

    def _min_step(i, m):
        return jnp.minimum(m, tsrow_v[pl.ds(i * 16, 16)])

    m = lax.fori_loop(0, _SEQ // 16, _min_step,
                      jnp.full((16,), 2**31 - 1, jnp.int32))
    ii = lax.iota(jnp.int32, 16)
    dnums = lax.GatherDimensionNumbers(
        offset_dims=(), collapsed_slice_dims=(0,), start_index_map=(0,))
    for sh in (8, 4, 2, 1):
        perm = (ii + sh) & 15
        shuf = lax.gather(m, perm[:, None], dnums, slice_sizes=(1,),
                          mode=lax.GatherScatterMode.PROMISE_IN_BOUNDS)
        m = jnp.minimum(m, shuf)

    def _delta_step(i, carry):
        d = tsrow_v[pl.ds(rib0 + i * 16, 16)] - m
        out_v[pl.ds(i * 16, 16)] = jnp.clip(d, 0, 4999)
        return carry

    lax.fori_loop(0, _ROWS_PER_TILE // 16, _delta_step, 0)
    pltpu.sync_copy(out_v, delta_hbm.at[pl.ds(wid * _ROWS_PER_TILE,
                                              _ROWS_PER_TILE)])


def _sc_delta(ts_flat):
    mesh = plsc.VectorSubcoreMesh(core_axis_name="c", subcore_axis_name="s")
    k = functools.partial(
        pl.kernel,
        mesh=mesh,
        out_type=jax.ShapeDtypeStruct((ts_flat.shape[0],), jnp.int32),
        scratch_types=[
            pltpu.VMEM((_SEQ,), jnp.int32),
            pltpu.VMEM((_ROWS_PER_TILE,), jnp.int32),
        ],
    )(_sc_delta_body)
    return k(ts_flat)


def _pe_add_body(d_ref, x_ref, hicat_ref, locat_ref, o_ref):
    d = x_ref.shape[-1]
    half = _SEQ_BLK // 2
    kio = jax.lax.broadcasted_iota(jnp.int32, (half, _KPAD), 1)
    for r0 in (0, half):
        delta = d_ref[0, 0, pl.ds(r0, half)]
        hi = delta // _STEP
        lo = delta - hi * _STEP
        a = (hi[:, None] == kio).astype(jnp.bfloat16)
        b = (lo[:, None] == kio).astype(jnp.bfloat16)
        hsw = jnp.dot(a, hicat_ref[...], preferred_element_type=jnp.float32)
        uv = jnp.dot(b, locat_ref[...], preferred_element_type=jnp.float32)
        pe = hsw[:, :d] * uv[:, :d] + hsw[:, d:] * uv[:, d:]
        o_ref[0, pl.ds(r0, half), :] = x_ref[0, pl.ds(r0, half), :] + pe


def _pair_swap(t):
    n, d = t.shape
    return t.reshape(n, d // 2, 2)[:, :, ::-1].reshape(n, d)


def kernel(x, timesteps, pos_encoding):
    b, seq, one, d = x.shape

    x3 = x.reshape(b, seq, d)
    ts_flat = timesteps.reshape(b * seq).astype(jnp.int32)

    delta = _sc_delta(ts_flat).reshape(b, 1, seq)

    n_hi = (pos_encoding.shape[0] + _STEP - 1) // _STEP
    hi_t = pos_encoding[:: _STEP]
    lo_t = pos_encoding[:_STEP]
    hi_sw = _pair_swap(hi_t)
    lo_sw = _pair_swap(lo_t)
    even = (jnp.arange(d) % 2 == 0)[None, :]
    u_t = jnp.where(even, lo_sw, lo_t)
    v_t = jnp.where(even, lo_t, -lo_sw)

    def _pad(t, rows):
        return jnp.pad(t, ((0, _KPAD - rows), (0, 0)))

    hicat = jnp.concatenate([_pad(hi_t, n_hi), _pad(hi_sw, n_hi)], axis=1)
    locat = jnp.concatenate([_pad(u_t, _STEP), _pad(v_t, _STEP)], axis=1)
    hicat = hicat.astype(jnp.bfloat16)
    locat = locat.astype(jnp.bfloat16)

    n_s = seq // _SEQ_BLK
    out = pl.pallas_call(
        _pe_add_body,
        grid=(b, n_s),
        in_specs=[
            pl.BlockSpec((1, 1, _SEQ_BLK), lambda i, j: (i, 0, j)),
            pl.BlockSpec((1, _SEQ_BLK, d), lambda i, j: (i, j, 0)),
            pl.BlockSpec((_KPAD, 2 * d), lambda i, j: (0, 0)),
            pl.BlockSpec((_KPAD, 2 * d), lambda i, j: (0, 0)),
        ],
        out_specs=pl.BlockSpec((1, _SEQ_BLK, d), lambda i, j: (i, j, 0)),
        out_shape=jax.ShapeDtypeStruct((b, seq, d), x.dtype),
    )(delta, x3, hicat, locat)
    return out.reshape(b, seq, one, d)
